# baseline (speedup 1.0000x reference)
import jax
import jax.numpy as jnp
from jax.experimental import pallas as pl
from jax.experimental.pallas import tpu as pltpu
from jax.experimental.pallas import tpu_sc as plsc

Nb, Na, Nnbh = 2, 5000, 16
NIN, NF, NOUT, NG = 128, 128, 128, 64

A_BLOCK = 200
R_BLOCK = A_BLOCK * Nnbh
GATHER_WINDOW = 256
SC_UNITS = 32
_LN2 = 0.6931471805599453
_LOG2E = 1.4426950408889634


def _ssp(v):
    t = jnp.exp(-jnp.abs(v))
    return jnp.maximum(v, 0.0) + jnp.log1p(t) - _LN2


def _sc_gather(table, idx_pad):
    n_idx = idx_pad.shape[0]
    c = table.shape[1]
    idx2 = idx_pad.reshape(1, n_idx)
    mesh = plsc.VectorSubcoreMesh(core_axis_name="c", subcore_axis_name="s")

    @pl.kernel(out_type=jax.ShapeDtypeStruct((n_idx, c), table.dtype), mesh=mesh)
    def gather_kernel(x_hbm, i_hbm, o_hbm):
        def body(i_vmem, o_vmem):
            pltpu.sync_copy(x_hbm.at[i_vmem.at[0]], o_vmem)

        pltpu.emit_pipeline(
            body,
            grid=(n_idx // GATHER_WINDOW,),
            in_specs=[pl.BlockSpec((1, GATHER_WINDOW), index_map=lambda i: (0, i))],
            out_specs=[pl.BlockSpec((GATHER_WINDOW, c), index_map=lambda i: (i, 0))],
            core_axis_name=("c", "s"),
            dimension_semantics=(pltpu.PARALLEL,),
        )(i_hbm, o_hbm)

    return gather_kernel(table, idx2)


def _fused_body(xg_ref, mask_ref, fs_ref, fp0_ref, fp1_ref, fp2_ref,
                wf1s_ref, bf1s_ref, wf2s_ref, bf2s_ref,
                wf1p_ref, bf1p_ref, wf2p_ref, bf2p_ref,
                ws_ref, wp_ref, wout_ref, bout_ref, o_ref):
    f32 = jnp.float32
    xg = xg_ref[...]
    mask3 = mask_ref[...][:, :, None]

    gs = jnp.dot(xg, ws_ref[...], preferred_element_type=f32)
    gp = jnp.dot(xg, wp_ref[...], preferred_element_type=f32)
    gs3 = gs.reshape(A_BLOCK, Nnbh, NF) * mask3
    gp3 = gp.reshape(A_BLOCK, Nnbh, NF) * mask3

    hs = _ssp(jnp.dot(fs_ref[...], wf1s_ref[...], preferred_element_type=f32)
              + bf1s_ref[...])
    Hs = jnp.dot(hs, wf2s_ref[...], preferred_element_type=f32) + bf2s_ref[...]
    ys = jnp.sum(gs3 * Hs.reshape(A_BLOCK, Nnbh, NF), axis=1)

    yp = jnp.zeros((A_BLOCK, NF), f32)
    for fpk_ref in (fp0_ref, fp1_ref, fp2_ref):
        hk = _ssp(jnp.dot(fpk_ref[...], wf1p_ref[...], preferred_element_type=f32)
                  + bf1p_ref[...])
        Hk = jnp.dot(hk, wf2p_ref[...], preferred_element_type=f32) + bf2p_ref[...]
        Sk = jnp.sum(gp3 * Hk.reshape(A_BLOCK, Nnbh, NF), axis=1)
        yp = yp + Sk * Sk

    y = ys + yp
    o_ref[...] = jnp.dot(y, wout_ref[...], preferred_element_type=f32) + bout_ref[...]


def _fused_specs():
    def full(shape):
        return pl.BlockSpec(shape, lambda i: (0,) * len(shape))

    in_specs = [
        pl.BlockSpec((R_BLOCK, NIN), lambda i: (i, 0)),
        pl.BlockSpec((A_BLOCK, Nnbh), lambda i: (i, 0)),
        pl.BlockSpec((R_BLOCK, NG), lambda i: (i, 0)),
        pl.BlockSpec((R_BLOCK, NG), lambda i: (i, 0)),
        pl.BlockSpec((R_BLOCK, NG), lambda i: (i, 0)),
        pl.BlockSpec((R_BLOCK, NG), lambda i: (i, 0)),
        full((NG, NF)), full((1, NF)), full((NF, NF)), full((1, NF)),
        full((NG, NF)), full((1, NF)), full((NF, NF)), full((1, NF)),
        full((NIN, NF)), full((NIN, NF)), full((NF, NOUT)), full((1, NOUT)),
    ]
    out_spec = pl.BlockSpec((A_BLOCK, NOUT), lambda i: (i, 0))
    grid = (Na // A_BLOCK,)
    return grid, in_specs, out_spec


def kernel(x, r_ij, neighbors, pairwise_mask, fsblock_ij, fpblock_ij,
           Wf1_s, bf1_s, Wf2_s, bf2_s, Wf1_p, bf1_p, Wf2_p, bf2_p,
           W_s, W_p, W_out, b_out):
    grid, in_specs, out_spec = _fused_specs()
    pad = (-(Na * Nnbh)) % (GATHER_WINDOW * SC_UNITS)
    zpad = jnp.zeros((pad,), jnp.int32)

    outs = []
    for b in range(Nb):
        idx_b = jnp.concatenate([neighbors[b].reshape(-1), zpad])
        xg_b = _sc_gather(x[b], idx_b)
        mask_b = pairwise_mask[b]
        fs_b = fsblock_ij[b].reshape(Na * Nnbh, NG)
        fp_b = [fpblock_ij[b, :, :, k, :].reshape(Na * Nnbh, NG) for k in range(3)]
        y_b = pl.pallas_call(
            _fused_body,
            grid=grid,
            in_specs=in_specs,
            out_specs=out_spec,
            out_shape=jax.ShapeDtypeStruct((Na, NOUT), jnp.float32),
        )(xg_b, mask_b, fs_b, fp_b[0], fp_b[1], fp_b[2],
          Wf1_s, bf1_s.reshape(1, NF), Wf2_s, bf2_s.reshape(1, NF),
          Wf1_p, bf1_p.reshape(1, NF), Wf2_p, bf2_p.reshape(1, NF),
          W_s, W_p, W_out, b_out.reshape(1, NOUT))
        outs.append(y_b)
    return jnp.stack(outs)


# speedup vs baseline: 1.4634x; 1.4634x over previous
import jax
import jax.numpy as jnp
from jax import lax
from jax.experimental import pallas as pl
from jax.experimental.pallas import tpu as pltpu
from jax.experimental.pallas import tpu_sc as plsc

Nb, Na, Nnbh = 2, 5000, 16
NIN, NF, NOUT, NG = 128, 128, 128, 64

GATHER_WINDOW = 256
SC_UNITS = 32
_LN2 = 0.6931471805599453


def _ssp(v):
    t = jnp.exp(-jnp.abs(v))
    return jnp.maximum(v, 0.0) + jnp.log1p(t) - _LN2


def _sc_gather(table, idx_pad):
    n_idx = idx_pad.shape[0]
    c = table.shape[1]
    idx2 = idx_pad.reshape(1, n_idx)
    mesh = plsc.VectorSubcoreMesh(core_axis_name="c", subcore_axis_name="s")

    @pl.kernel(out_type=jax.ShapeDtypeStruct((n_idx, c), table.dtype), mesh=mesh)
    def gather_kernel(x_hbm, i_hbm, o_hbm):
        def body(i_vmem, o_vmem):
            pltpu.sync_copy(x_hbm.at[i_vmem.at[0]], o_vmem)

        pltpu.emit_pipeline(
            body,
            grid=(n_idx // GATHER_WINDOW,),
            in_specs=[pl.BlockSpec((1, GATHER_WINDOW), index_map=lambda i: (0, i))],
            out_specs=[pl.BlockSpec((GATHER_WINDOW, c), index_map=lambda i: (i, 0))],
            core_axis_name=("c", "s"),
            dimension_semantics=(pltpu.PARALLEL,),
        )(i_hbm, o_hbm)

    return gather_kernel(table, idx2)


def _fused_body(xg_ref, mask_ref, fs_ref, fp_ref,
                w1st_ref, b1s_ref, w2st_ref, b2s_ref,
                w1pt_ref, b1p_ref, w2pt_ref, b2p_ref,
                wst_ref, wpt_ref, woutt_ref, bout_ref, o_ref,
                ys_acc, yp0_acc, yp1_acc, yp2_acc):
    f32 = jnp.float32
    n = pl.program_id(0)

    xgt = jnp.transpose(xg_ref[...])
    gst = jnp.dot(wst_ref[...], xgt, preferred_element_type=f32)
    gpt = jnp.dot(wpt_ref[...], xgt, preferred_element_type=f32)
    mask = mask_ref[0]

    fsb = fs_ref[0]
    hs = _ssp(jnp.dot(w1st_ref[...], fsb, preferred_element_type=f32)
              + b1s_ref[...])
    hst = jnp.dot(w2st_ref[...], hs, preferred_element_type=f32) + b2s_ref[...]
    s_term = mask * (gst * hst)

    fpb = fp_ref[0]
    p_terms = []
    for k in range(3):
        hk = _ssp(jnp.dot(w1pt_ref[...], fpb[k * NG:(k + 1) * NG],
                          preferred_element_type=f32) + b1p_ref[...])
        hkt = jnp.dot(w2pt_ref[...], hk, preferred_element_type=f32) + b2p_ref[...]
        p_terms.append(mask * (gpt * hkt))

    @pl.when(n == 0)
    def _():
        ys_acc[...] = s_term
        yp0_acc[...] = p_terms[0]
        yp1_acc[...] = p_terms[1]
        yp2_acc[...] = p_terms[2]

    @pl.when(n > 0)
    def _():
        ys_acc[...] += s_term
        yp0_acc[...] += p_terms[0]
        yp1_acc[...] += p_terms[1]
        yp2_acc[...] += p_terms[2]

    @pl.when(n == Nnbh - 1)
    def _():
        y0, y1, y2 = yp0_acc[...], yp1_acc[...], yp2_acc[...]
        y = ys_acc[...] + y0 * y0 + y1 * y1 + y2 * y2
        out_t = jnp.dot(woutt_ref[...], y, preferred_element_type=f32) + bout_ref[...]
        o_ref[...] = jnp.transpose(out_t)


def _fused_call():
    def full(shape):
        return pl.BlockSpec(shape, lambda n: (0,) * len(shape))

    in_specs = [
        pl.BlockSpec((Na, NIN), lambda n: (n, 0)),
        pl.BlockSpec((1, 1, Na), lambda n: (n, 0, 0)),
        pl.BlockSpec((1, NG, Na), lambda n: (n, 0, 0)),
        pl.BlockSpec((1, 3 * NG, Na), lambda n: (n, 0, 0)),
        full((NF, NG)), full((NF, 1)), full((NF, NF)), full((NF, 1)),
        full((NF, NG)), full((NF, 1)), full((NF, NF)), full((NF, 1)),
        full((NF, NIN)), full((NF, NIN)), full((NOUT, NF)), full((NOUT, 1)),
    ]
    out_spec = pl.BlockSpec((Na, NOUT), lambda n: (0, 0))
    scratch = [pltpu.VMEM((NF, Na), jnp.float32) for _ in range(4)]
    return pl.pallas_call(
        _fused_body,
        grid=(Nnbh,),
        in_specs=in_specs,
        out_specs=out_spec,
        out_shape=jax.ShapeDtypeStruct((Na, NOUT), jnp.float32),
        scratch_shapes=scratch,
    )


def kernel(x, r_ij, neighbors, pairwise_mask, fsblock_ij, fpblock_ij,
           Wf1_s, bf1_s, Wf2_s, bf2_s, Wf1_p, bf1_p, Wf2_p, bf2_p,
           W_s, W_p, W_out, b_out):
    pad = (-(Na * Nnbh)) % (GATHER_WINDOW * SC_UNITS)
    zpad = jnp.zeros((pad,), jnp.int32)

    w1st = Wf1_s.T
    w2st = Wf2_s.T
    w1pt = Wf1_p.T
    w2pt = Wf2_p.T
    wst = W_s.T
    wpt = W_p.T
    woutt = W_out.T
    b1s = bf1_s.reshape(NF, 1)
    b2s = bf2_s.reshape(NF, 1)
    b1p = bf1_p.reshape(NF, 1)
    b2p = bf2_p.reshape(NF, 1)
    bout = b_out.reshape(NOUT, 1)

    call = _fused_call()
    outs = []
    for b in range(Nb):
        idx_b = jnp.concatenate(
            [jnp.swapaxes(neighbors[b], 0, 1).reshape(-1), zpad])
        xg_b = _sc_gather(x[b], idx_b)
        mask_t = jnp.swapaxes(pairwise_mask[b], 0, 1).reshape(Nnbh, 1, Na)
        fs_t = jnp.transpose(fsblock_ij[b], (1, 2, 3, 0)).reshape(Nnbh, NG, Na)
        fp_t = jnp.transpose(fpblock_ij[b], (1, 2, 3, 0)).reshape(Nnbh, 3 * NG, Na)
        y_b = call(xg_b, mask_t, fs_t, fp_t,
                   w1st, b1s, w2st, b2s,
                   w1pt, b1p, w2pt, b2p,
                   wst, wpt, woutt, bout)
        outs.append(y_b)
    return jnp.stack(outs)
